# baseline (speedup 1.0000x reference)
import functools

import jax
import jax.numpy as jnp
from jax import lax
from jax.experimental import pallas as pl
from jax.experimental.pallas import tpu as pltpu
from jax.experimental.pallas import tpu_sc as plsc

B = 16384
D = 32
NW = 32
BPW = B // NW
CH = 128
NCH = BPW // CH
NG = BPW // 16

_mesh = plsc.VectorSubcoreMesh(core_axis_name="c", subcore_axis_name="s")


def _rsqrt(x):
    i = plsc.bitcast(x, jnp.int32)
    y = plsc.bitcast(jnp.int32(0x5F3759DF) - (i >> 1), jnp.float32)
    for _ in range(3):
        y = y * (1.5 - 0.5 * x * y * y)
    return y


@functools.partial(
    pl.kernel,
    mesh=_mesh,
    out_type=jax.ShapeDtypeStruct((B,), jnp.float32),
    compiler_params=pltpu.CompilerParams(
        needs_layout_passes=False, use_tc_tiling_on_sc=False
    ),
    scratch_types=[
        pltpu.VMEM((NCH, CH), jnp.int32),
        pltpu.VMEM((NCH, CH), jnp.int32),
        pltpu.VMEM((BPW, D), jnp.float32),
        pltpu.VMEM((BPW, D), jnp.float32),
        pltpu.VMEM((BPW, D), jnp.float32),
        pltpu.VMEM((BPW, D), jnp.float32),
        pltpu.VMEM((D,), jnp.float32),
        pltpu.VMEM((D,), jnp.float32),
        pltpu.VMEM((BPW,), jnp.float32),
        pltpu.SemaphoreType.DMA,
    ],
)
def _sc_kernel(src_hbm, anc_hbm, emb1_hbm, emb2_hbm, rel1_hbm, rel2_hbm,
               out_hbm, sidx, aidx, s1r, a1r, s2r, a2r, rel1v, rel2v,
               outv, sem):
    wid = lax.axis_index("s") * 2 + lax.axis_index("c")
    rowbase = wid * NCH
    pltpu.sync_copy(src_hbm.at[pl.ds(rowbase, NCH)], sidx)
    pltpu.sync_copy(anc_hbm.at[pl.ds(rowbase, NCH)], aidx)
    pltpu.sync_copy(rel1_hbm, rel1v)
    pltpu.sync_copy(rel2_hbm, rel2v)

    copies = []
    for j in range(NCH):
        dst = pl.ds(j * CH, CH)
        copies.append(pltpu.async_copy(emb1_hbm.at[sidx.at[j]], s1r.at[dst], sem))
        copies.append(pltpu.async_copy(emb1_hbm.at[aidx.at[j]], a1r.at[dst], sem))
        copies.append(pltpu.async_copy(emb2_hbm.at[sidx.at[j]], s2r.at[dst], sem))
        copies.append(pltpu.async_copy(emb2_hbm.at[aidx.at[j]], a2r.at[dst], sem))
    for c in copies:
        c.wait()

    iota16 = lax.iota(jnp.int32, 16)
    zero = jnp.zeros((16,), jnp.float32)
    r1a = rel1v[pl.ds(0, 16)]
    r1b = rel1v[pl.ds(16, 16)]
    r2a = rel2v[pl.ds(0, 16)]
    r2b = rel2v[pl.ds(16, 16)]

    def group(g, carry):
        q = g * 16 + iota16
        num1 = ss1 = aa1 = zero
        num2 = ss2 = aa2 = zero
        for d in range(D):
            dv = jnp.full((16,), d, jnp.int32)
            s1 = plsc.load_gather(s1r, [q, dv])
            a1 = plsc.load_gather(a1r, [q, dv])
            s2 = plsc.load_gather(s2r, [q, dv])
            a2 = plsc.load_gather(a2r, [q, dv])
            r1 = r1a[d] if d < 16 else r1b[d - 16]
            r2 = r2a[d] if d < 16 else r2b[d - 16]
            v1 = s1 * r1
            v2 = s2 * r2
            num1 = num1 + v1 * a1
            ss1 = ss1 + v1 * v1
            aa1 = aa1 + a1 * a1
            num2 = num2 + v2 * a2
            ss2 = ss2 + v2 * v2
            aa2 = aa2 + a2 * a2
        dim1 = num1 * _rsqrt(jnp.maximum(ss1 * aa1, 1e-16))
        dim2 = num2 * _rsqrt(jnp.maximum(ss2 * aa2, 1e-16))
        res = 1.0 - (1.0 - dim1) * (1.0 - dim2)
        outv[pl.ds(g * 16, 16)] = res
        return carry

    lax.fori_loop(0, NG, group, 0)
    pltpu.sync_copy(outv, out_hbm.at[pl.ds(wid * BPW, BPW)])


def kernel(source_nodes, anchor_nodes, emb1, emb2, rel1, rel2):
    src = source_nodes.astype(jnp.int32).reshape(NW * NCH, CH)
    anc = anchor_nodes.astype(jnp.int32).reshape(NW * NCH, CH)
    return _sc_kernel(src, anc, emb1, emb2, rel1, rel2)


# speedup vs baseline: 4.1396x; 4.1396x over previous
import functools

import jax
import jax.numpy as jnp
from jax import lax
from jax.experimental import pallas as pl
from jax.experimental.pallas import tpu as pltpu
from jax.experimental.pallas import tpu_sc as plsc

B = 16384
D = 32
V = 1000000
NT = 16
QPT = B // NT
NCH = QPT // 128
NGRP = QPT // 16

_mesh = plsc.VectorSubcoreMesh(core_axis_name="c", subcore_axis_name="s")


def _rsqrt(x):
    i = plsc.bitcast(x, jnp.int32)
    y = plsc.bitcast(jnp.int32(0x5F3759DF) - (i >> 1), jnp.float32)
    for _ in range(3):
        y = y * (1.5 - 0.5 * x * y * y)
    return y


@functools.partial(
    pl.kernel,
    mesh=_mesh,
    out_type=jax.ShapeDtypeStruct((2, B), jnp.float32),
    compiler_params=pltpu.CompilerParams(needs_layout_passes=False),
    scratch_types=[
        pltpu.VMEM((NCH, 128), jnp.int32),
        pltpu.VMEM((NCH, 128), jnp.int32),
        pltpu.VMEM_SHARED((V,), jnp.float32),
        pltpu.VMEM((QPT,), jnp.float32),
        pltpu.VMEM((QPT,), jnp.float32),
        pltpu.VMEM((QPT,), jnp.float32),
        pltpu.VMEM((QPT,), jnp.float32),
        pltpu.VMEM((QPT,), jnp.float32),
        pltpu.VMEM((D,), jnp.float32),
        pltpu.SemaphoreType.DMA,
        pltpu.SemaphoreType.DMA,
    ],
)
def _sc_kernel(src_hbm, anc_hbm, e1t_hbm, e2t_hbm, rel_hbm,
               out_hbm, sidx, aidx, rowa, sval, aval, numv, ssv, aav,
               relv, sema, gsem):
    sc = lax.axis_index("c")
    tid = lax.axis_index("s")
    rowbase = tid * NCH

    pltpu.sync_copy(src_hbm.at[pl.ds(rowbase, NCH)], sidx)
    pltpu.sync_copy(anc_hbm.at[pl.ds(rowbase, NCH)], aidx)
    pltpu.sync_copy(rel_hbm.at[sc], relv)

    def zero_body(g, carry):
        q = pl.ds(g * 16, 16)
        z = jnp.zeros((16,), jnp.float32)
        numv[q] = z
        ssv[q] = z
        aav[q] = z
        return carry

    lax.fori_loop(0, NGRP, zero_body, 0)

    r_lo = relv[pl.ds(0, 16)]
    r_hi = relv[pl.ds(16, 16)]

    def run_table(tbl):
        @pl.when(tid == 0)
        def _():
            pltpu.async_copy(tbl.at[0], rowa, sema)

        def phase(f, carry):
            @pl.when(tid == 0)
            def _():
                pltpu.make_async_copy(tbl.at[0], rowa, sema).wait()

            plsc.subcore_barrier()

            copies = []
            for ch in range(NCH):
                dst = pl.ds(ch * 128, 128)
                copies.append(
                    pltpu.async_copy(rowa.at[sidx.at[ch]], sval.at[dst], gsem))
                copies.append(
                    pltpu.async_copy(rowa.at[aidx.at[ch]], aval.at[dst], gsem))
            for cp in copies:
                cp.wait()

            plsc.subcore_barrier()

            @pl.when((tid == 0) & (f + 1 < D))
            def _():
                pltpu.async_copy(tbl.at[f + 1], rowa, sema)

            fv = jnp.zeros((16,), jnp.int32) + (f & 15)
            rb = jnp.where(
                jnp.zeros((16,), jnp.int32) + f < 16,
                r_lo.at[fv].get(mode="promise_in_bounds"),
                r_hi.at[fv].get(mode="promise_in_bounds"),
            )

            def acc_body(g, carry):
                q = pl.ds(g * 16, 16)
                s = sval[q]
                a = aval[q]
                v = s * rb
                numv[q] = numv[q] + v * a
                ssv[q] = ssv[q] + v * v
                aav[q] = aav[q] + a * a
                return carry

            lax.fori_loop(0, NGRP, acc_body, 0)
            return carry

        lax.fori_loop(0, D, phase, 0)

    @pl.when(sc == 0)
    def _():
        run_table(e1t_hbm)

    @pl.when(sc == 1)
    def _():
        run_table(e2t_hbm)

    def fin_body(g, carry):
        q = pl.ds(g * 16, 16)
        num = numv[q]
        den2 = jnp.maximum(ssv[q] * aav[q], 1e-16)
        sval[q] = num * _rsqrt(den2)
        return carry

    lax.fori_loop(0, NGRP, fin_body, 0)
    pltpu.sync_copy(sval, out_hbm.at[sc, pl.ds(tid * QPT, QPT)])


def _combine_body(d_ref, o_ref):
    d1 = d_ref[0]
    d2 = d_ref[1]
    o_ref[...] = 1.0 - (1.0 - d1) * (1.0 - d2)


_combine = pl.pallas_call(
    _combine_body,
    out_shape=jax.ShapeDtypeStruct((128, 128), jnp.float32),
)


def kernel(source_nodes, anchor_nodes, emb1, emb2, rel1, rel2):
    src = source_nodes.astype(jnp.int32).reshape(NT * NCH, 128)
    anc = anchor_nodes.astype(jnp.int32).reshape(NT * NCH, 128)
    relb = jnp.stack([rel1, rel2])
    dims = _sc_kernel(src, anc, emb1.T, emb2.T, relb)
    return _combine(dims.reshape(2, 128, 128)).reshape(B)


# speedup vs baseline: 5.2280x; 1.2629x over previous
import functools

import jax
import jax.numpy as jnp
from jax import lax
from jax.experimental import pallas as pl
from jax.experimental.pallas import tpu as pltpu
from jax.experimental.pallas import tpu_sc as plsc

B = 16384
D = 32
V = 1000000
NT = 16
QPT = B // NT
NCH = QPT // 128
NGC = 128 // 16

_mesh = plsc.VectorSubcoreMesh(core_axis_name="c", subcore_axis_name="s")


def _rsqrt(x):
    i = plsc.bitcast(x, jnp.int32)
    y = plsc.bitcast(jnp.int32(0x5F3759DF) - (i >> 1), jnp.float32)
    for _ in range(3):
        y = y * (1.5 - 0.5 * x * y * y)
    return y


@functools.partial(
    pl.kernel,
    mesh=_mesh,
    out_type=jax.ShapeDtypeStruct((2, B), jnp.float32),
    compiler_params=pltpu.CompilerParams(needs_layout_passes=False),
    scratch_types=[
        pltpu.VMEM((NCH, 128), jnp.int32),
        pltpu.VMEM((NCH, 128), jnp.int32),
        pltpu.VMEM_SHARED((V,), jnp.float32),
        pltpu.VMEM_SHARED((V,), jnp.float32),
        pltpu.VMEM((2, 128), jnp.float32),
        pltpu.VMEM((2, 128), jnp.float32),
        pltpu.VMEM((QPT,), jnp.float32),
        pltpu.VMEM((QPT,), jnp.float32),
        pltpu.VMEM((QPT,), jnp.float32),
        pltpu.VMEM((D,), jnp.float32),
        pltpu.SemaphoreType.DMA,
        pltpu.SemaphoreType.DMA,
        pltpu.SemaphoreType.DMA,
        pltpu.SemaphoreType.DMA,
    ],
)
def _sc_kernel(src_hbm, anc_hbm, e1t_hbm, e2t_hbm, rel_hbm,
               out_hbm, sidx, aidx, rowa, rowb, sbank, abank, numv, ssv, aav,
               relv, sema, semb, gsem0, gsem1):
    sc = lax.axis_index("c")
    tid = lax.axis_index("s")
    rowbase = tid * NCH

    pltpu.sync_copy(src_hbm.at[pl.ds(rowbase, NCH)], sidx)
    pltpu.sync_copy(anc_hbm.at[pl.ds(rowbase, NCH)], aidx)
    pltpu.sync_copy(rel_hbm.at[sc], relv)

    def zero_body(g, carry):
        q = pl.ds(g * 16, 16)
        z = jnp.zeros((16,), jnp.float32)
        numv[q] = z
        ssv[q] = z
        aav[q] = z
        return carry

    lax.fori_loop(0, QPT // 16, zero_body, 0)

    r_lo = relv[pl.ds(0, 16)]
    r_hi = relv[pl.ds(16, 16)]
    gsems = (gsem0, gsem1)

    def run_table(tbl):
        @pl.when(tid == 0)
        def _():
            pltpu.async_copy(tbl.at[0], rowa, sema)
            pltpu.async_copy(tbl.at[1], rowb, semb)

        def phase(f, buf, sem):
            @pl.when(tid == 0)
            def _():
                pltpu.make_async_copy(tbl.at[0], buf, sem).wait()

            plsc.subcore_barrier()

            fv = jnp.zeros((16,), jnp.int32) + (f & 15)
            rb = jnp.where(
                jnp.zeros((16,), jnp.int32) + f < 16,
                r_lo.at[fv].get(mode="promise_in_bounds"),
                r_hi.at[fv].get(mode="promise_in_bounds"),
            )

            def issue(ch, b):
                pltpu.async_copy(
                    buf.at[sidx.at[ch]], sbank.at[b], gsems[b])
                pltpu.async_copy(
                    buf.at[aidx.at[ch]], abank.at[b], gsems[b])

            def wait(b):
                pltpu.make_async_copy(
                    buf.at[sidx.at[0]], sbank.at[b], gsems[b]).wait()
                pltpu.make_async_copy(
                    buf.at[aidx.at[0]], abank.at[b], gsems[b]).wait()

            def acc_chunk(ch, b):
                for g in range(NGC):
                    q = pl.ds(ch * 128 + g * 16, 16)
                    s = sbank[b, pl.ds(g * 16, 16)]
                    a = abank[b, pl.ds(g * 16, 16)]
                    v = s * rb
                    numv[q] = numv[q] + v * a
                    ssv[q] = ssv[q] + v * v
                    aav[q] = aav[q] + a * a

            issue(0, 0)
            for ch in range(1, NCH):
                issue(ch, ch % 2)
                wait((ch - 1) % 2)
                acc_chunk(ch - 1, (ch - 1) % 2)
            wait((NCH - 1) % 2)

            plsc.subcore_barrier()

            @pl.when((tid == 0) & (f + 2 < D))
            def _():
                pltpu.async_copy(tbl.at[f + 2], buf, sem)

            acc_chunk(NCH - 1, (NCH - 1) % 2)

        def step(k, carry):
            phase(2 * k, rowa, sema)
            phase(2 * k + 1, rowb, semb)
            return carry

        lax.fori_loop(0, D // 2, step, 0)

    @pl.when(sc == 0)
    def _():
        run_table(e1t_hbm)

    @pl.when(sc == 1)
    def _():
        run_table(e2t_hbm)

    def fin_body(g, carry):
        q = pl.ds(g * 16, 16)
        num = numv[q]
        den2 = jnp.maximum(ssv[q] * aav[q], 1e-16)
        numv[q] = num * _rsqrt(den2)
        return carry

    lax.fori_loop(0, QPT // 16, fin_body, 0)
    pltpu.sync_copy(numv, out_hbm.at[sc, pl.ds(tid * QPT, QPT)])


def _combine_body(d_ref, o_ref):
    d1 = d_ref[0]
    d2 = d_ref[1]
    o_ref[...] = 1.0 - (1.0 - d1) * (1.0 - d2)


_combine = pl.pallas_call(
    _combine_body,
    out_shape=jax.ShapeDtypeStruct((128, 128), jnp.float32),
)


def kernel(source_nodes, anchor_nodes, emb1, emb2, rel1, rel2):
    src = source_nodes.astype(jnp.int32).reshape(NT * NCH, 128)
    anc = anchor_nodes.astype(jnp.int32).reshape(NT * NCH, 128)
    relb = jnp.stack([rel1, rel2])
    dims = _sc_kernel(src, anc, emb1.T, emb2.T, relb)
    return _combine(dims.reshape(2, 128, 128)).reshape(B)
